# baseline (speedup 1.0000x reference)
import functools

import jax
import jax.numpy as jnp
from jax import lax
from jax.experimental import pallas as pl
from jax.experimental.pallas import tpu as pltpu
from jax.experimental.pallas import tpu_sc as plsc

NUM_CLASSES = 10
NUM_CORES = 1
NUM_SUBCORES = 16
NUM_WORKERS = NUM_CORES * NUM_SUBCORES
LANES = 16


def _onehot_t_sc(labels):
    batch = labels.shape[0]
    b_per_w = batch // NUM_WORKERS

    mesh = plsc.VectorSubcoreMesh(
        core_axis_name="c", subcore_axis_name="s", num_cores=NUM_CORES)

    @functools.partial(
        pl.kernel,
        mesh=mesh,
        out_type=jax.ShapeDtypeStruct((NUM_CLASSES, batch), jnp.float32),
        scratch_types=[
            pltpu.VMEM((b_per_w,), jnp.int32),
            pltpu.VMEM((NUM_CLASSES, b_per_w), jnp.float32),
            pltpu.SemaphoreType.DMA,
        ],
        compiler_params=pltpu.CompilerParams(
            needs_layout_passes=False,
            skip_device_barrier=True,
            disable_bounds_checks=True,
            disable_semaphore_checks=True,
        ),
    )
    def k(idx_hbm, out_hbm, idx_v, cols_v, sem):
        wid = lax.axis_index("s") * NUM_CORES + lax.axis_index("c")
        base = wid * b_per_w
        quarter = b_per_w // 4
        pltpu.sync_copy(idx_hbm.at[pl.ds(base, b_per_w)], idx_v)

        ones = jnp.ones((LANES,), jnp.float32)
        zeros = jnp.zeros((LANES,), jnp.float32)

        copies = []
        for q in range(4):
            lo = q * quarter

            @plsc.parallel_loop(lo // LANES, (lo + quarter) // LANES,
                                unroll=2)
            def body(i):
                lbl = idx_v[pl.ds(i * LANES, LANES)]
                for c in range(NUM_CLASSES):
                    cols_v[c, pl.ds(i * LANES, LANES)] = jnp.where(
                        lbl == c, ones, zeros)

            copies.append(pltpu.async_copy(
                cols_v.at[:, pl.ds(lo, quarter)],
                out_hbm.at[:, pl.ds(base + lo, quarter)], sem))
        for cp in copies:
            cp.wait()

    return k(labels)


def kernel(images, labels):
    del images
    return _onehot_t_sc(labels.astype(jnp.int32)).T


# speedup vs baseline: 1.0108x; 1.0108x over previous
import functools

import jax
import jax.numpy as jnp
from jax import lax
from jax.experimental import pallas as pl
from jax.experimental.pallas import tpu as pltpu
from jax.experimental.pallas import tpu_sc as plsc

NUM_CLASSES = 10
NUM_CORES = 1
NUM_SUBCORES = 16
NUM_WORKERS = NUM_CORES * NUM_SUBCORES
LANES = 16


def _onehot_t_sc(labels):
    batch = labels.shape[0]
    b_per_w = batch // NUM_WORKERS

    mesh = plsc.VectorSubcoreMesh(
        core_axis_name="c", subcore_axis_name="s", num_cores=NUM_CORES)

    @functools.partial(
        pl.kernel,
        mesh=mesh,
        out_type=jax.ShapeDtypeStruct((NUM_CLASSES, batch), jnp.float32),
        scratch_types=[
            pltpu.VMEM((b_per_w,), jnp.int32),
            pltpu.VMEM((NUM_CLASSES, b_per_w), jnp.float32),
            pltpu.SemaphoreType.DMA,
        ],
        compiler_params=pltpu.CompilerParams(
            needs_layout_passes=False,
            skip_device_barrier=True,
            disable_bounds_checks=True,
            disable_semaphore_checks=True,
        ),
    )
    def k(idx_hbm, out_hbm, idx_v, cols_v, sem):
        wid = lax.axis_index("s") * NUM_CORES + lax.axis_index("c")
        base = wid * b_per_w
        half = b_per_w // 2
        pltpu.sync_copy(idx_hbm.at[pl.ds(base, b_per_w)], idx_v)

        ones = jnp.ones((LANES,), jnp.float32)
        zeros = jnp.zeros((LANES,), jnp.float32)

        @plsc.parallel_loop(0, half // LANES, unroll=2)
        def body_lo(i):
            lbl = idx_v[pl.ds(i * LANES, LANES)]
            for c in range(NUM_CLASSES):
                cols_v[c, pl.ds(i * LANES, LANES)] = jnp.where(
                    lbl == c, ones, zeros)

        first = pltpu.async_copy(
            cols_v.at[:, pl.ds(0, half)],
            out_hbm.at[:, pl.ds(base, half)], sem)

        @plsc.parallel_loop(half // LANES, b_per_w // LANES, unroll=2)
        def body_hi(i):
            lbl = idx_v[pl.ds(i * LANES, LANES)]
            for c in range(NUM_CLASSES):
                cols_v[c, pl.ds(i * LANES, LANES)] = jnp.where(
                    lbl == c, ones, zeros)

        pltpu.sync_copy(cols_v.at[:, pl.ds(half, half)],
                        out_hbm.at[:, pl.ds(base + half, half)])
        first.wait()

    return k(labels)


def kernel(images, labels):
    del images
    return _onehot_t_sc(labels.astype(jnp.int32)).T
